# baseline (speedup 1.0000x reference)
import jax
import jax.numpy as jnp
from jax.experimental import pallas as pl


def _pool_body(x_ref, o_ref):
    x = x_ref[...]
    cells = []
    for ph in range(4):
        band = jnp.sum(x[:, :, 6 * ph:6 * ph + 6, :], axis=2)
        for pw in range(4):
            cells.append(jnp.sum(band[:, :, 6 * pw:6 * pw + 6],
                                 axis=2, keepdims=True))
    o_ref[...] = jnp.concatenate(cells, axis=2)


def _head_body(f_ref, w1_ref, b1_ref, w2_ref, b2_ref, g_ref, i_ref):
    scale = jnp.float32(1.0 / 36.0)
    h = jax.lax.dot_general(f_ref[...] * scale, w1_ref[...],
                            (((1,), (1,)), ((), ())),
                            preferred_element_type=jnp.float32) + b1_ref[...]
    h = jnp.maximum(h, 0.0)
    logits = jax.lax.dot_general(h, w2_ref[...],
                                 (((1,), (1,)), ((), ())),
                                 preferred_element_type=jnp.float32) + b2_ref[...]
    B, E = logits.shape
    lane = jax.lax.broadcasted_iota(jnp.int32, (B, E), 1)
    m1 = jnp.max(logits, axis=-1, keepdims=True)
    i1 = jnp.min(jnp.where(logits == m1, lane, E), axis=-1, keepdims=True)
    masked = jnp.where(lane == i1, -jnp.inf, logits)
    m2 = jnp.max(masked, axis=-1, keepdims=True)
    i2 = jnp.min(jnp.where(masked == m2, lane, E), axis=-1, keepdims=True)
    e2 = jnp.exp(m2 - m1)
    g1 = 1.0 / (1.0 + e2)
    g2 = e2 / (1.0 + e2)
    g_ref[...] = (jnp.where(lane == i1, g1, 0.0)
                  + jnp.where(lane == i2, g2, 0.0))
    i_ref[...] = jnp.where(lane == 0, i1, 0) + jnp.where(lane == 1, i2, 0)


def kernel(x, W1, b1, W2, b2):
    B, C, H, W = x.shape
    E = W2.shape[0]
    OH = OW = 4
    pooled = pl.pallas_call(
        _pool_body,
        grid=(B,),
        in_specs=[pl.BlockSpec((1, C, H, W), lambda i: (i, 0, 0, 0))],
        out_specs=pl.BlockSpec((1, C, OH * OW), lambda i: (i, 0, 0)),
        out_shape=jax.ShapeDtypeStruct((B, C, OH * OW), jnp.float32),
    )(x)
    flat = pooled.reshape(B, C * OH * OW)
    gates, ipad = pl.pallas_call(
        _head_body,
        out_shape=[jax.ShapeDtypeStruct((B, E), jnp.float32),
                   jax.ShapeDtypeStruct((B, E), jnp.int32)],
    )(flat, W1, b1.reshape(1, -1), W2, b2.reshape(1, -1))
    return gates, ipad[:, :2]


# speedup vs baseline: 1.1725x; 1.1725x over previous
import jax
import jax.numpy as jnp
from jax.experimental import pallas as pl


def _pool_body(x_ref, o_ref):
    x = x_ref[...]
    cells = []
    for ph in range(4):
        band = jnp.sum(x[:, 6 * ph:6 * ph + 6, :], axis=1)
        for pw in range(4):
            cells.append(jnp.sum(band[:, 6 * pw:6 * pw + 6],
                                 axis=1, keepdims=True))
    o_ref[...] = jnp.concatenate(cells, axis=1)


def _head_body(f_ref, w1_ref, b1_ref, w2_ref, b2_ref, g_ref, i_ref):
    scale = jnp.float32(1.0 / 36.0)
    h = jax.lax.dot_general(f_ref[...] * scale, w1_ref[...],
                            (((1,), (1,)), ((), ())),
                            preferred_element_type=jnp.float32) + b1_ref[...]
    h = jnp.maximum(h, 0.0)
    logits = jax.lax.dot_general(h, w2_ref[...],
                                 (((1,), (1,)), ((), ())),
                                 preferred_element_type=jnp.float32) + b2_ref[...]
    B, E = logits.shape
    lane = jax.lax.broadcasted_iota(jnp.int32, (B, E), 1)
    m1 = jnp.max(logits, axis=-1, keepdims=True)
    i1 = jnp.min(jnp.where(logits == m1, lane, E), axis=-1, keepdims=True)
    masked = jnp.where(lane == i1, -jnp.inf, logits)
    m2 = jnp.max(masked, axis=-1, keepdims=True)
    i2 = jnp.min(jnp.where(masked == m2, lane, E), axis=-1, keepdims=True)
    e2 = jnp.exp(m2 - m1)
    g1 = 1.0 / (1.0 + e2)
    g2 = e2 / (1.0 + e2)
    g_ref[...] = (jnp.where(lane == i1, g1, 0.0)
                  + jnp.where(lane == i2, g2, 0.0))
    i_ref[...] = jnp.where(lane == 0, i1, 0) + jnp.where(lane == 1, i2, 0)


def kernel(x, W1, b1, W2, b2):
    B, C, H, W = x.shape
    E = W2.shape[0]
    OH = OW = 4
    ROWS = B * C
    xr = x.reshape(ROWS, H, W)
    BLK = 1024
    pooled = pl.pallas_call(
        _pool_body,
        grid=(ROWS // BLK,),
        in_specs=[pl.BlockSpec((BLK, H, W), lambda i: (i, 0, 0))],
        out_specs=pl.BlockSpec((BLK, OH * OW), lambda i: (i, 0)),
        out_shape=jax.ShapeDtypeStruct((ROWS, OH * OW), jnp.float32),
    )(xr)
    flat = pooled.reshape(B, C * OH * OW)
    gates, ipad = pl.pallas_call(
        _head_body,
        out_shape=[jax.ShapeDtypeStruct((B, E), jnp.float32),
                   jax.ShapeDtypeStruct((B, E), jnp.int32)],
    )(flat, W1, b1.reshape(1, -1), W2, b2.reshape(1, -1))
    return gates, ipad[:, :2]
